# baseline (speedup 1.0000x reference)
import jax
import jax.numpy as jnp
from jax import lax
from jax.experimental import pallas as pl
from jax.experimental.pallas import tpu as pltpu
from jax.experimental.pallas import tpu_sc as plsc

N = 10000
D = 128
NPAD = 10112
NC = 2
NS = 16
RPT = NPAD // NS
CHUNK = 128
EPAD = 327680
NCHUNKS = EPAD // CHUNK
CPT = NCHUNKS // (NC * NS)
NB = 2

_MESH = dict(core_axis_name="c", subcore_axis_name="s",
             num_cores=NC, num_subcores=NS)


def _unpack_chunk(pidx_v, j, sst_v, dst_v, slot):
    for u in range(CHUNK // 16):
        v = pidx_v[j, pl.ds(u * 16, 16)]
        sst_v[slot, pl.ds(u * 16, 16)] = v & 0xFFFF
        dst_v[slot, pl.ds(u * 16, 16)] = lax.shift_right_logical(v, 16)


def _deg_sc(pidx_hbm, z_hbm, ones_hbm, out_hbm, pidx_v, dst_v, ones_v,
            acc_sh):
    c = lax.axis_index("c")
    s = lax.axis_index("s")
    tid = c * NS + s
    pltpu.sync_copy(z_hbm.at[pl.ds(s * RPT, RPT)],
                    acc_sh.at[pl.ds(s * RPT, RPT)])
    pltpu.sync_copy(ones_hbm, ones_v)
    pltpu.sync_copy(pidx_hbm.at[tid], pidx_v)
    plsc.subcore_barrier()

    def body(j, carry):
        for u in range(CHUNK // 16):
            v = pidx_v[j, pl.ds(u * 16, 16)]
            dst_v[pl.ds(u * 16, 16)] = lax.shift_right_logical(v, 16)
        pltpu.sync_copy(ones_v, acc_sh.at[dst_v], add=True)
        return carry

    lax.fori_loop(0, CPT, body, 0)
    plsc.subcore_barrier()
    pltpu.sync_copy(acc_sh.at[pl.ds(s * RPT, RPT)],
                    out_hbm.at[c, pl.ds(s * RPT, RPT)])


def _acc_sc(pidx_hbm, h_hbm, z_hbm, out_hbm,
            pidx_v, sst_v, dst_v, rows_v, g0, g1, ssem, acc_sh):
    c = lax.axis_index("c")
    s = lax.axis_index("s")
    tid = c * NS + s
    gsems = (g0, g1)
    pltpu.sync_copy(z_hbm.at[pl.ds(s * RPT, RPT)],
                    acc_sh.at[pl.ds(s * RPT, RPT)])
    pltpu.sync_copy(pidx_hbm.at[tid], pidx_v)
    plsc.subcore_barrier()

    for i in range(NB):
        _unpack_chunk(pidx_v, i, sst_v, dst_v, i)
        pltpu.async_copy(h_hbm.at[sst_v.at[i]], rows_v.at[i], gsems[i])

    def body(k, carry):
        for i in range(NB):
            j = k * NB + i
            pltpu.make_async_copy(h_hbm.at[sst_v.at[i]], rows_v.at[i],
                                  gsems[i]).wait()
            pltpu.async_copy(rows_v.at[i], acc_sh.at[dst_v.at[i]], ssem,
                             add=True)
            pltpu.make_async_copy(rows_v.at[i], acc_sh.at[dst_v.at[i]],
                                  ssem).wait()

            @pl.when(j + NB < CPT)
            def _():
                _unpack_chunk(pidx_v, j + NB, sst_v, dst_v, i)
                pltpu.async_copy(h_hbm.at[sst_v.at[i]], rows_v.at[i],
                                 gsems[i])
        return carry

    lax.fori_loop(0, CPT // NB, body, 0)
    plsc.subcore_barrier()
    pltpu.sync_copy(acc_sh.at[pl.ds(s * RPT, RPT)],
                    out_hbm.at[c, pl.ds(s * RPT, RPT)])


def _run_deg(pidx_p, z128, ones128):
    return pl.kernel(
        _deg_sc,
        out_type=jax.ShapeDtypeStruct((NC, NPAD, D), jnp.float32),
        mesh=plsc.VectorSubcoreMesh(**_MESH),
        scratch_types=[
            pltpu.VMEM((CPT, CHUNK), jnp.int32),
            pltpu.VMEM((CHUNK,), jnp.int32),
            pltpu.VMEM((CHUNK, D), jnp.float32),
            pltpu.VMEM_SHARED((NPAD, D), jnp.float32),
        ],
    )(pidx_p, z128, ones128)


def _run_acc(pidx_p, h, z128):
    return pl.kernel(
        _acc_sc,
        out_type=jax.ShapeDtypeStruct((NC, NPAD, D), jnp.float32),
        mesh=plsc.VectorSubcoreMesh(**_MESH),
        scratch_types=[
            pltpu.VMEM((CPT, CHUNK), jnp.int32),
            pltpu.VMEM((NB, CHUNK), jnp.int32),
            pltpu.VMEM((NB, CHUNK), jnp.int32),
            pltpu.VMEM((NB, CHUNK, D), jnp.float32),
            pltpu.SemaphoreType.DMA,
            pltpu.SemaphoreType.DMA,
            pltpu.SemaphoreType.DMA,
            pltpu.VMEM_SHARED((NPAD, D), jnp.float32),
        ],
    )(pidx_p, h, z128)


BR = 400
GRID = N // BR


def _dinv(d0_ref, d1_ref):
    deg = d0_ref[0, :, 0:1] + d1_ref[0, :, 0:1] + 1.0
    return lax.rsqrt(deg)


def _mm(a, w):
    return lax.dot_general(a, w, (((1,), (0,)), ((), ())),
                           precision=lax.Precision.HIGHEST,
                           preferred_element_type=jnp.float32)


def _h1p_body(x_ref, w_ref, d0_ref, d1_ref, o_ref):
    o_ref[...] = _mm(x_ref[...], w_ref[...]) * _dinv(d0_ref, d1_ref)


def _h2p_body(p0_ref, p1_ref, h_ref, d0_ref, d1_ref, b_ref, w_ref, o_ref):
    dinv = _dinv(d0_ref, d1_ref)
    acc = p0_ref[0] + p1_ref[0] + h_ref[...]
    z = jnp.maximum(acc * dinv + b_ref[...], 0.0)
    o_ref[...] = _mm(z, w_ref[...]) * dinv


def _out_body(q0_ref, q1_ref, h_ref, d0_ref, d1_ref, b_ref, o_ref):
    dinv = _dinv(d0_ref, d1_ref)
    o_ref[...] = (q0_ref[0] + q1_ref[0] + h_ref[...]) * dinv + b_ref[...]


_row_spec = pl.BlockSpec((BR, D), lambda i: (i, 0))
_w_spec = pl.BlockSpec((D, D), lambda i: (0, 0))
_b_spec = pl.BlockSpec((1, D), lambda i: (0, 0))
_deg0_spec = pl.BlockSpec((1, BR, D), lambda i: (0, i, 0))
_deg1_spec = pl.BlockSpec((1, BR, D), lambda i: (1, i, 0))
_p0_spec = pl.BlockSpec((1, BR, D), lambda i: (0, i, 0))
_p1_spec = pl.BlockSpec((1, BR, D), lambda i: (1, i, 0))
_out_sds = jax.ShapeDtypeStruct((N, D), jnp.float32)


def kernel(x, edge_index, W1, b1, W2, b2):
    e = edge_index.shape[1]
    pad = EPAD - e
    packed = edge_index[0] | (edge_index[1] << 16)
    packed = jnp.concatenate(
        [packed, jnp.full((pad,), N << 16, jnp.int32)]
    ).reshape(NC * NS, CPT, CHUNK)
    z128 = jnp.zeros((NPAD, D), jnp.float32)
    ones128 = jnp.ones((CHUNK, D), jnp.float32)
    b1r = b1.reshape(1, D)
    b2r = b2.reshape(1, D)

    degp = _run_deg(packed, z128, ones128)

    h1p = pl.pallas_call(
        _h1p_body,
        grid=(GRID,),
        in_specs=[_row_spec, _w_spec, _deg0_spec, _deg1_spec],
        out_specs=_row_spec,
        out_shape=_out_sds,
    )(x, W1, degp, degp)

    p = _run_acc(packed, h1p, z128)

    h2p = pl.pallas_call(
        _h2p_body,
        grid=(GRID,),
        in_specs=[_p0_spec, _p1_spec, _row_spec, _deg0_spec, _deg1_spec,
                  _b_spec, _w_spec],
        out_specs=_row_spec,
        out_shape=_out_sds,
    )(p, p, h1p, degp, degp, b1r, W2)

    q = _run_acc(packed, h2p, z128)

    out = pl.pallas_call(
        _out_body,
        grid=(GRID,),
        in_specs=[_p0_spec, _p1_spec, _row_spec, _deg0_spec, _deg1_spec,
                  _b_spec],
        out_specs=_row_spec,
        out_shape=_out_sds,
    )(q, q, h2p, degp, degp, b2r)

    return out


# speedup vs baseline: 1.2400x; 1.2400x over previous
import jax
import jax.numpy as jnp
from jax import lax
from jax.experimental import pallas as pl
from jax.experimental.pallas import tpu as pltpu
from jax.experimental.pallas import tpu_sc as plsc

N = 10000
D = 128
NPAD = 10112
NC = 2
NS = 16
RPT = NPAD // NS
CHUNK = 128
EPAD = 327680
NCHUNKS = EPAD // CHUNK
BLK = 16
NBLK = NCHUNKS // BLK
CPT = NCHUNKS // (NC * NS)
NB = 2
CPT0 = 144
CPT1 = 16
NBLK0 = CPT0 // BLK
NBLK1 = CPT1 // BLK

_MESH = dict(core_axis_name="c", subcore_axis_name="s",
             num_cores=NC, num_subcores=NS)


def _unpack_chunk(pidx_v, bi, off, sst_v, dst_v, slot):
    for u in range(CHUNK // 16):
        v = pidx_v[bi, off, pl.ds(u * 16, 16)]
        sst_v[slot, pl.ds(u * 16, 16)] = v & 0xFFFF
        dst_v[slot, pl.ds(u * 16, 16)] = lax.shift_right_logical(v, 16)


def _deg_sc(pidx_hbm, z_hbm, ones_hbm, out_hbm, pidx_v, dst_v, ones_v,
            acc_sh):
    c = lax.axis_index("c")
    s = lax.axis_index("s")
    tid = c * NS + s
    pltpu.sync_copy(z_hbm.at[pl.ds(s * RPT, RPT)],
                    acc_sh.at[pl.ds(s * RPT, RPT)])
    pltpu.sync_copy(ones_hbm, ones_v)
    pltpu.sync_copy(pidx_hbm.at[pl.ds(tid * (CPT // BLK), CPT // BLK)],
                    pidx_v)
    plsc.subcore_barrier()

    def body(j, carry):
        bi = lax.div(j, BLK)
        off = lax.rem(j, BLK)
        for u in range(CHUNK // 16):
            v = pidx_v[bi, off, pl.ds(u * 16, 16)]
            dst_v[pl.ds(u * 16, 16)] = lax.shift_right_logical(v, 16)
        pltpu.sync_copy(ones_v, acc_sh.at[dst_v], add=True)
        return carry

    lax.fori_loop(0, CPT, body, 0)
    plsc.subcore_barrier()
    pltpu.sync_copy(acc_sh.at[pl.ds(s * RPT, RPT)],
                    out_hbm.at[c, pl.ds(s * RPT, RPT)])


def _acc_sc(pidx_hbm, h_hbm, z_hbm, out_hbm,
            blk_v, sst_v, dst_v, rows_v, g0, g1, ssem, bsem, acc_sh):
    c = lax.axis_index("c")
    s = lax.axis_index("s")
    gsems = (g0, g1)
    nblk = jnp.where(c == 0, NBLK0, NBLK1)
    ncpt = nblk * BLK
    blk0 = jnp.where(c == 0, s * NBLK0, NS * NBLK0 + s * NBLK1)

    pltpu.sync_copy(z_hbm.at[pl.ds(s * RPT, RPT)],
                    acc_sh.at[pl.ds(s * RPT, RPT)])
    pltpu.async_copy(pidx_hbm.at[blk0], blk_v.at[0], bsem)
    plsc.subcore_barrier()

    pltpu.make_async_copy(pidx_hbm.at[blk0], blk_v.at[0], bsem).wait()

    @pl.when(nblk > 1)
    def _():
        pltpu.async_copy(pidx_hbm.at[blk0 + 1], blk_v.at[1], bsem)

    for i in range(NB):
        _unpack_chunk(blk_v, 0, i, sst_v, dst_v, i)
        pltpu.async_copy(h_hbm.at[sst_v.at[i]], rows_v.at[i], gsems[i])

    def body(j, carry):
        i = lax.rem(j, NB)

        def phase(i):
            pltpu.make_async_copy(h_hbm.at[sst_v.at[i]], rows_v.at[i],
                                  gsems[i]).wait()
            pltpu.async_copy(rows_v.at[i], acc_sh.at[dst_v.at[i]], ssem,
                             add=True)
            pltpu.make_async_copy(rows_v.at[i], acc_sh.at[dst_v.at[i]],
                                  ssem).wait()
            nxt = j + NB

            @pl.when(nxt < ncpt)
            def _():
                b = lax.div(nxt, BLK)
                off = lax.rem(nxt, BLK)

                @pl.when(off == 0)
                def _():
                    pltpu.make_async_copy(pidx_hbm.at[blk0], blk_v.at[0],
                                          bsem).wait()

                    @pl.when(b + 1 < nblk)
                    def _():
                        pltpu.async_copy(
                            pidx_hbm.at[blk0 + b + 1],
                            blk_v.at[lax.rem(b + 1, 2)], bsem)

                _unpack_chunk(blk_v, lax.rem(b, 2), off, sst_v, dst_v, i)
                pltpu.async_copy(h_hbm.at[sst_v.at[i]], rows_v.at[i],
                                 gsems[i])

        @pl.when(i == 0)
        def _():
            phase(0)

        @pl.when(i == 1)
        def _():
            phase(1)

        return carry

    lax.fori_loop(0, ncpt, body, 0)
    plsc.subcore_barrier()
    pltpu.sync_copy(acc_sh.at[pl.ds(s * RPT, RPT)],
                    out_hbm.at[c, pl.ds(s * RPT, RPT)])


def _run_deg(pidx_p, z128, ones128):
    return pl.kernel(
        _deg_sc,
        out_type=jax.ShapeDtypeStruct((NC, NPAD, D), jnp.float32),
        mesh=plsc.VectorSubcoreMesh(**_MESH),
        scratch_types=[
            pltpu.VMEM((CPT // BLK, BLK, CHUNK), jnp.int32),
            pltpu.VMEM((CHUNK,), jnp.int32),
            pltpu.VMEM((CHUNK, D), jnp.float32),
            pltpu.VMEM_SHARED((NPAD, D), jnp.float32),
        ],
    )(pidx_p, z128, ones128)


def _run_acc(pidx_p, h, z128):
    return pl.kernel(
        _acc_sc,
        out_type=jax.ShapeDtypeStruct((NC, NPAD, D), jnp.float32),
        mesh=plsc.VectorSubcoreMesh(**_MESH),
        scratch_types=[
            pltpu.VMEM((2, BLK, CHUNK), jnp.int32),
            pltpu.VMEM((NB, CHUNK), jnp.int32),
            pltpu.VMEM((NB, CHUNK), jnp.int32),
            pltpu.VMEM((NB, CHUNK, D), jnp.float32),
            pltpu.SemaphoreType.DMA,
            pltpu.SemaphoreType.DMA,
            pltpu.SemaphoreType.DMA,
            pltpu.SemaphoreType.DMA,
            pltpu.VMEM_SHARED((NPAD, D), jnp.float32),
        ],
    )(pidx_p, h, z128)


BR = 400
GRID = N // BR


def _dinv(d0_ref, d1_ref):
    deg = d0_ref[0, :, 0:1] + d1_ref[0, :, 0:1] + 1.0
    return lax.rsqrt(deg)


def _mm(a, w):
    return lax.dot_general(a, w, (((1,), (0,)), ((), ())),
                           precision=lax.Precision.HIGHEST,
                           preferred_element_type=jnp.float32)


def _h1p_body(x_ref, w_ref, d0_ref, d1_ref, o_ref):
    o_ref[...] = _mm(x_ref[...], w_ref[...]) * _dinv(d0_ref, d1_ref)


def _h2p_body(p0_ref, p1_ref, h_ref, d0_ref, d1_ref, b_ref, w_ref, o_ref):
    dinv = _dinv(d0_ref, d1_ref)
    acc = p0_ref[0] + p1_ref[0] + h_ref[...]
    z = jnp.maximum(acc * dinv + b_ref[...], 0.0)
    o_ref[...] = _mm(z, w_ref[...]) * dinv


def _out_body(q0_ref, q1_ref, h_ref, d0_ref, d1_ref, b_ref, o_ref):
    dinv = _dinv(d0_ref, d1_ref)
    o_ref[...] = (q0_ref[0] + q1_ref[0] + h_ref[...]) * dinv + b_ref[...]


_row_spec = pl.BlockSpec((BR, D), lambda i: (i, 0))
_w_spec = pl.BlockSpec((D, D), lambda i: (0, 0))
_b_spec = pl.BlockSpec((1, D), lambda i: (0, 0))
_deg0_spec = pl.BlockSpec((1, BR, D), lambda i: (0, i, 0))
_deg1_spec = pl.BlockSpec((1, BR, D), lambda i: (1, i, 0))
_p0_spec = pl.BlockSpec((1, BR, D), lambda i: (0, i, 0))
_p1_spec = pl.BlockSpec((1, BR, D), lambda i: (1, i, 0))
_out_sds = jax.ShapeDtypeStruct((N, D), jnp.float32)


def kernel(x, edge_index, W1, b1, W2, b2):
    e = edge_index.shape[1]
    pad = EPAD - e
    packed = edge_index[0] | (edge_index[1] << 16)
    packed = jnp.concatenate(
        [packed, jnp.full((pad,), N << 16, jnp.int32)]
    ).reshape(NBLK, BLK, CHUNK)
    z128 = jnp.zeros((NPAD, D), jnp.float32)
    ones128 = jnp.ones((CHUNK, D), jnp.float32)
    b1r = b1.reshape(1, D)
    b2r = b2.reshape(1, D)

    degp = _run_deg(packed, z128, ones128)

    h1p = pl.pallas_call(
        _h1p_body,
        grid=(GRID,),
        in_specs=[_row_spec, _w_spec, _deg0_spec, _deg1_spec],
        out_specs=_row_spec,
        out_shape=_out_sds,
    )(x, W1, degp, degp)

    p = _run_acc(packed, h1p, z128)

    h2p = pl.pallas_call(
        _h2p_body,
        grid=(GRID,),
        in_specs=[_p0_spec, _p1_spec, _row_spec, _deg0_spec, _deg1_spec,
                  _b_spec, _w_spec],
        out_specs=_row_spec,
        out_shape=_out_sds,
    )(p, p, h1p, degp, degp, b1r, W2)

    q = _run_acc(packed, h2p, z128)

    out = pl.pallas_call(
        _out_body,
        grid=(GRID,),
        in_specs=[_p0_spec, _p1_spec, _row_spec, _deg0_spec, _deg1_spec,
                  _b_spec],
        out_specs=_row_spec,
        out_shape=_out_sds,
    )(q, q, h2p, degp, degp, b2r)

    return out
